# baseline (speedup 1.0000x reference)
import functools

import jax
import jax.numpy as jnp
from jax import lax
from jax.experimental import pallas as pl
from jax.experimental.pallas import tpu as pltpu
from jax.experimental.pallas import tpu_sc as plsc

_VOCAB = 100000
_D = 128
_B = 4096
_S = 50
_NC, _NS = 2, 16
_NW = _NC * _NS
_FLAT = _B * _S
_LPW = _FLAT // _NW
_C = 128
_NCHUNK = _LPW // _C
_NBUF = 4
_LEAD = 2


def _build_lookup():
    mesh = plsc.VectorSubcoreMesh(core_axis_name="c", subcore_axis_name="s")

    scratch = [
        pltpu.VMEM((_NCHUNK, _C), jnp.int32),
        pltpu.VMEM((_NBUF, _C, _D), jnp.float32),
    ] + [pltpu.SemaphoreType.DMA] * (2 * _NBUF)

    @functools.partial(
        pl.kernel,
        mesh=mesh,
        out_type=jax.ShapeDtypeStruct((_FLAT, _D), jnp.float32),
        scratch_types=scratch,
    )
    def lookup(idx_hbm, table_hbm, out_hbm, idx_v, rows, *sems):
        sem_g = sems[:_NBUF]
        sem_s = sems[_NBUF:]
        wid = lax.axis_index("s") * _NC + lax.axis_index("c")
        base = wid * _LPW
        pltpu.sync_copy(idx_hbm.at[wid], idx_v)

        def gather(c, b):
            pltpu.async_copy(table_hbm.at[idx_v.at[c]], rows.at[b], sem_g[b])

        def gather_wait(b):
            pltpu.make_async_copy(
                table_hbm.at[idx_v.at[0]], rows.at[b], sem_g[b]
            ).wait()

        def store(c, b):
            pltpu.async_copy(
                rows.at[b], out_hbm.at[pl.ds(base + c * _C, _C)], sem_s[b]
            )

        def store_wait(b):
            pltpu.make_async_copy(
                rows.at[b], out_hbm.at[pl.ds(0, _C)], sem_s[b]
            ).wait()

        for c in range(_LEAD):
            gather(c, c)

        for c in range(_NCHUNK):
            b = c % _NBUF
            if c + _LEAD < _NCHUNK:
                fb = (c + _LEAD) % _NBUF
                if c >= _NBUF - _LEAD:
                    store_wait(fb)
                gather(c + _LEAD, fb)
            gather_wait(b)
            store(c, b)

        for b in range(_NBUF):
            store_wait(b)

    return lookup


_lookup = _build_lookup()


def kernel(inputs, embedding_table):
    idx = inputs.reshape(_NW, _NCHUNK, _C)
    return _lookup(idx, embedding_table).reshape(_B, _S, _D)


# speedup vs baseline: 1.7794x; 1.7794x over previous
import functools

import jax
import jax.numpy as jnp
from jax import lax
from jax.experimental import pallas as pl
from jax.experimental.pallas import tpu as pltpu
from jax.experimental.pallas import tpu_sc as plsc

_VOCAB = 100000
_D = 128
_B = 4096
_S = 50
_NC, _NS = 2, 16
_NW = _NC * _NS
_BPW = _B // _NW
_G = 2
_NGROUPS = _BPW // _G
_NBUF = 8
_LEAD = 4


def _build_lookup():
    mesh = plsc.VectorSubcoreMesh(core_axis_name="c", subcore_axis_name="s")

    scratch = [
        pltpu.VMEM((_BPW, _S), jnp.int32),
        pltpu.VMEM((_NBUF, _G, _S, _D), jnp.float32),
    ] + [pltpu.SemaphoreType.DMA] * (2 * _NBUF)

    @functools.partial(
        pl.kernel,
        mesh=mesh,
        out_type=jax.ShapeDtypeStruct((_B, _S, _D), jnp.float32),
        scratch_types=scratch,
    )
    def lookup(idx_hbm, table_hbm, out_hbm, idx_v, rows, *sems):
        sem_g = sems[:_NBUF]
        sem_s = sems[_NBUF:]
        wid = lax.axis_index("s") * _NC + lax.axis_index("c")
        base = wid * _BPW
        pltpu.sync_copy(idx_hbm.at[wid], idx_v)

        def gather_group(g, b):
            for i in range(_G):
                pltpu.async_copy(
                    table_hbm.at[idx_v.at[g * _G + i]], rows.at[b].at[i], sem_g[b]
                )

        def gather_wait(b):
            for i in range(_G):
                pltpu.make_async_copy(
                    table_hbm.at[idx_v.at[0]], rows.at[b].at[i], sem_g[b]
                ).wait()

        def store(g, b):
            pltpu.async_copy(
                rows.at[b], out_hbm.at[pl.ds(base + g * _G, _G)], sem_s[b]
            )

        def store_wait(b):
            pltpu.make_async_copy(
                rows.at[b], out_hbm.at[pl.ds(0, _G)], sem_s[b]
            ).wait()

        for g in range(_LEAD):
            gather_group(g, g)

        for g in range(_NGROUPS):
            b = g % _NBUF
            if g + _LEAD < _NGROUPS:
                fb = (g + _LEAD) % _NBUF
                if g >= _NBUF - _LEAD:
                    store_wait(fb)
                gather_group(g + _LEAD, fb)
            gather_wait(b)
            store(g, b)

        for b in range(_NBUF):
            store_wait(b)

    return lookup


_lookup = _build_lookup()


def kernel(inputs, embedding_table):
    idx = inputs.reshape(_NW, _BPW, _S)
    return _lookup(idx, embedding_table)


# speedup vs baseline: 1.7813x; 1.0011x over previous
import functools

import jax
import jax.numpy as jnp
from jax import lax
from jax.experimental import pallas as pl
from jax.experimental.pallas import tpu as pltpu
from jax.experimental.pallas import tpu_sc as plsc

_VOCAB = 100000
_D = 128
_B = 4096
_S = 50
_NC, _NS = 2, 16
_NW = _NC * _NS
_BPW = _B // _NW
_G = 2
_NGROUPS = _BPW // _G
_NBUF = 8
_LEAD = 6


def _build_lookup():
    mesh = plsc.VectorSubcoreMesh(core_axis_name="c", subcore_axis_name="s")

    scratch = [
        pltpu.VMEM((_BPW, _S), jnp.int32),
        pltpu.VMEM((_NBUF, _G, _S, _D), jnp.float32),
    ] + [pltpu.SemaphoreType.DMA] * (2 * _NBUF)

    @functools.partial(
        pl.kernel,
        mesh=mesh,
        out_type=jax.ShapeDtypeStruct((_B, _S, _D), jnp.float32),
        scratch_types=scratch,
    )
    def lookup(idx_hbm, table_hbm, out_hbm, idx_v, rows, *sems):
        sem_g = sems[:_NBUF]
        sem_s = sems[_NBUF:]
        wid = lax.axis_index("s") * _NC + lax.axis_index("c")
        base = wid * _BPW
        pltpu.sync_copy(idx_hbm.at[wid], idx_v)

        def gather_group(g, b):
            for i in range(_G):
                pltpu.async_copy(
                    table_hbm.at[idx_v.at[g * _G + i]], rows.at[b].at[i], sem_g[b]
                )

        def gather_wait(b):
            for i in range(_G):
                pltpu.make_async_copy(
                    table_hbm.at[idx_v.at[0]], rows.at[b].at[i], sem_g[b]
                ).wait()

        def store(g, b):
            pltpu.async_copy(
                rows.at[b], out_hbm.at[pl.ds(base + g * _G, _G)], sem_s[b]
            )

        def store_wait(b):
            pltpu.make_async_copy(
                rows.at[b], out_hbm.at[pl.ds(0, _G)], sem_s[b]
            ).wait()

        for g in range(_LEAD):
            gather_group(g, g)

        for g in range(_NGROUPS):
            b = g % _NBUF
            if g + _LEAD < _NGROUPS:
                fb = (g + _LEAD) % _NBUF
                if g >= _NBUF - _LEAD:
                    store_wait(fb)
                gather_group(g + _LEAD, fb)
            gather_wait(b)
            store(g, b)

        for b in range(_NBUF):
            store_wait(b)

    return lookup


_lookup = _build_lookup()


def kernel(inputs, embedding_table):
    idx = inputs.reshape(_NW, _BPW, _S)
    return _lookup(idx, embedding_table)
